# baseline (speedup 1.0000x reference)
import functools

import jax
import jax.numpy as jnp
from jax.experimental import pallas as pl
from jax.experimental.pallas import tpu as pltpu

B = 32
D = 256
SLOTS = 4096
SBLK = 256
NBLK = SLOTS // SBLK
INV_SQRT_D = 1.0 / 16.0


def _pre_kernel(s_ref, wvec_ref, gate_ref, Wq_ref, Wl_ref, bl_ref, Wk_ref,
                Wv_ref, gw_ref, q_ref, wval_ref, c_ref):
    s = s_ref[...]
    logits = jax.lax.dot_general(s, Wl_ref[...], (((1,), (1,)), ((), ())),
                                 preferred_element_type=jnp.float32)
    logits = logits + bl_ref[...][None, :]
    m = jnp.max(logits, axis=-1, keepdims=True)
    e = jnp.exp(logits - m)
    w = e / jnp.sum(e, axis=-1, keepdims=True)
    gw_ref[...] = gate_ref[...] * w
    q = jax.lax.dot_general(s, Wq_ref[...], (((1,), (1,)), ((), ())),
                            preferred_element_type=jnp.float32)
    q_ref[...] = q
    wvec = wvec_ref[...]
    wk = jax.lax.dot_general(wvec, Wk_ref[...], (((1,), (1,)), ((), ())),
                             preferred_element_type=jnp.float32)
    wval_ref[...] = jax.lax.dot_general(wvec, Wv_ref[...], (((1,), (1,)), ((), ())),
                                        preferred_element_type=jnp.float32)
    c = jnp.sum(q * wk, axis=-1, keepdims=True)
    c_ref[...] = jnp.broadcast_to(c, (B, 128))


def _att_kernel(q_ref, gw_ref, c_ref, mk_ref, att_ref):
    q = q_ref[...]
    gw = gw_ref[...]
    c = c_ref[...][:, :1]
    a0 = jnp.sum(q[:, None, :] * mk_ref[...], axis=-1)
    att_ref[...] = (a0 * (1.0 - gw) + gw * c) * INV_SQRT_D


def _softmax_kernel(att_ref, gw_ref, wr_ref, swr_ref):
    att = att_ref[...]
    gw = gw_ref[...]
    m = jnp.max(att, axis=-1, keepdims=True)
    e = jnp.exp(att - m)
    denom = jnp.sum(e, axis=-1, keepdims=True)
    p = e / denom
    wr_ref[...] = p * (1.0 - gw)
    swr_ref[...] = jnp.broadcast_to(
        jnp.sum(p * gw, axis=-1, keepdims=True), (B, 128))


def _out_kernel(wr_ref, wval_ref, swr_ref, mv_ref, out_ref):
    i = pl.program_id(0)

    @pl.when(i == 0)
    def _():
        out_ref[...] = swr_ref[...][:, :1] * wval_ref[...]

    part = jnp.sum(wr_ref[...][:, :, None] * mv_ref[...], axis=1)
    out_ref[...] += part


def kernel(s, write_vec, mem_k, mem_v, gate, Wq, Wl, bl, Wk, Wv):
    f32 = jnp.float32
    gw, q, wval, c = pl.pallas_call(
        _pre_kernel,
        out_shape=(
            jax.ShapeDtypeStruct((B, SLOTS), f32),
            jax.ShapeDtypeStruct((B, D), f32),
            jax.ShapeDtypeStruct((B, D), f32),
            jax.ShapeDtypeStruct((B, 128), f32),
        ),
    )(s, write_vec, gate, Wq, Wl, bl, Wk, Wv)

    att = pl.pallas_call(
        _att_kernel,
        grid=(NBLK,),
        in_specs=[
            pl.BlockSpec((B, D), lambda i: (0, 0)),
            pl.BlockSpec((B, SBLK), lambda i: (0, i)),
            pl.BlockSpec((B, 128), lambda i: (0, 0)),
            pl.BlockSpec((B, SBLK, D), lambda i: (0, i, 0)),
        ],
        out_specs=pl.BlockSpec((B, SBLK), lambda i: (0, i)),
        out_shape=jax.ShapeDtypeStruct((B, SLOTS), f32),
    )(q, gw, c, mem_k)

    wr, swr = pl.pallas_call(
        _softmax_kernel,
        out_shape=(
            jax.ShapeDtypeStruct((B, SLOTS), f32),
            jax.ShapeDtypeStruct((B, 128), f32),
        ),
    )(att, gw)

    out = pl.pallas_call(
        _out_kernel,
        grid=(NBLK,),
        in_specs=[
            pl.BlockSpec((B, SBLK), lambda i: (0, i)),
            pl.BlockSpec((B, D), lambda i: (0, 0)),
            pl.BlockSpec((B, 128), lambda i: (0, 0)),
            pl.BlockSpec((B, SBLK, D), lambda i: (0, i, 0)),
        ],
        out_specs=pl.BlockSpec((B, D), lambda i: (0, 0)),
        out_shape=jax.ShapeDtypeStruct((B, D), f32),
    )(wr, wval, swr, mem_v)
    return out


# speedup vs baseline: 1.0279x; 1.0279x over previous
import functools

import jax
import jax.numpy as jnp
from jax.experimental import pallas as pl
from jax.experimental.pallas import tpu as pltpu

B = 32
D = 256
SLOTS = 4096
SBLK = 256
NBLK = SLOTS // SBLK
INV_SQRT_D = 1.0 / 16.0


def _fused_kernel(s_ref, wvec_ref, gate_ref, Wq_ref, Wl_ref, bl_ref, Wk_ref,
                  Wv_ref, mk_ref, mv_ref, out_ref,
                  gw_s, att_s, q_s, wval_s, c_s, swr_s, acc_s):
    g = pl.program_id(0)

    @pl.when(g == 0)
    def _prologue():
        s = s_ref[...]
        logits = jax.lax.dot_general(s, Wl_ref[...], (((1,), (1,)), ((), ())),
                                     preferred_element_type=jnp.float32)
        logits = logits + bl_ref[...][None, :]
        m = jnp.max(logits, axis=-1, keepdims=True)
        e = jnp.exp(logits - m)
        w = e / jnp.sum(e, axis=-1, keepdims=True)
        gw_s[...] = gate_ref[...] * w
        q = jax.lax.dot_general(s, Wq_ref[...], (((1,), (1,)), ((), ())),
                                preferred_element_type=jnp.float32)
        q_s[...] = q
        wvec = wvec_ref[...]
        wk = jax.lax.dot_general(wvec, Wk_ref[...], (((1,), (1,)), ((), ())),
                                 preferred_element_type=jnp.float32)
        wval_s[...] = jax.lax.dot_general(wvec, Wv_ref[...],
                                          (((1,), (1,)), ((), ())),
                                          preferred_element_type=jnp.float32)
        c_s[...] = jnp.broadcast_to(jnp.sum(q * wk, axis=-1, keepdims=True),
                                    (B, 128))

    @pl.when(g < NBLK)
    def _att_phase():
        i = g
        q = q_s[...]
        gw = gw_s[pl.ds(0, B), pl.ds(i * SBLK, SBLK)]
        c = c_s[...][:, :1]
        a0 = jnp.sum(q[:, None, :] * mk_ref[...], axis=-1)
        att_s[pl.ds(0, B), pl.ds(i * SBLK, SBLK)] = (
            (a0 * (1.0 - gw) + gw * c) * INV_SQRT_D)

    @pl.when(g == NBLK)
    def _softmax():
        att = att_s[...]
        gw = gw_s[...]
        m = jnp.max(att, axis=-1, keepdims=True)
        e = jnp.exp(att - m)
        denom = jnp.sum(e, axis=-1, keepdims=True)
        p = e / denom
        att_s[...] = p * (1.0 - gw)
        swr_s[...] = jnp.broadcast_to(
            jnp.sum(p * gw, axis=-1, keepdims=True), (B, 128))
        acc_s[...] = swr_s[...][:, :1] * wval_s[...]

    @pl.when(g >= NBLK)
    def _out_phase():
        i = g - NBLK
        wr = att_s[pl.ds(0, B), pl.ds(i * SBLK, SBLK)]
        acc_s[...] += jnp.sum(wr[:, :, None] * mv_ref[...], axis=1)

    @pl.when(g == 2 * NBLK - 1)
    def _epilogue():
        out_ref[...] = acc_s[...]


def kernel(s, write_vec, mem_k, mem_v, gate, Wq, Wl, bl, Wk, Wv):
    f32 = jnp.float32
    whole = lambda shape: pl.BlockSpec(shape, lambda g: tuple(0 for _ in shape))
    out = pl.pallas_call(
        _fused_kernel,
        grid=(2 * NBLK,),
        in_specs=[
            whole((B, D)),
            whole((B, D)),
            whole((B, 1)),
            whole((D, D)),
            whole((SLOTS, D)),
            whole((SLOTS,)),
            whole((D, D)),
            whole((D, D)),
            pl.BlockSpec((B, SBLK, D),
                         lambda g: (0, jnp.minimum(g, NBLK - 1), 0)),
            pl.BlockSpec((B, SBLK, D),
                         lambda g: (0, jnp.maximum(g - NBLK, 0), 0)),
        ],
        out_specs=pl.BlockSpec((B, D), lambda g: (0, 0)),
        out_shape=jax.ShapeDtypeStruct((B, D), f32),
        scratch_shapes=[
            pltpu.VMEM((B, SLOTS), f32),
            pltpu.VMEM((B, SLOTS), f32),
            pltpu.VMEM((B, D), f32),
            pltpu.VMEM((B, D), f32),
            pltpu.VMEM((B, 128), f32),
            pltpu.VMEM((B, 128), f32),
            pltpu.VMEM((B, D), f32),
        ],
    )(s, write_vec, gate, Wq, Wl, bl, Wk, Wv, mem_k, mem_v)
    return out
